# baseline (speedup 1.0000x reference)
import functools

import jax
import jax.numpy as jnp
from jax import lax
from jax.experimental import pallas as pl
from jax.experimental.pallas import tpu as pltpu
from jax.experimental.pallas import tpu_sc as plsc

NUM_EMB = 1_000_000
DIM = 128
NCB = 8
CBS = 256
SUB = 16
BATCH = 16384

_INFO = plsc.get_sparse_core_info()
NC, NS, L = _INFO.num_cores, _INFO.num_subcores, _INFO.num_lanes
NW = NC * NS
BPW = BATCH // NW
CHUNK = 128
NIC = BPW // CHUNK
NGC = BPW * NCB // CHUNK


def _pq_body(idx_hbm, cb_hbm, codes_hbm, out_hbm, idx_v, codes_v, g_v,
             rows_v, sem):
    wid = lax.axis_index("s") * NC + lax.axis_index("c")

    pltpu.sync_copy(idx_hbm.at[pl.ds(wid * NIC, NIC)], idx_v)

    handles = []
    for j in range(NIC):
        handles.append(
            pltpu.async_copy(codes_hbm.at[idx_v.at[j]],
                             codes_v.at[pl.ds(j * CHUNK, CHUNK)], sem))
    for h in handles:
        h.wait()

    lane = lax.iota(jnp.int32, L)

    def g_chunk(t, carry):
        i = t >> 2
        iv = jnp.full((L,), 0, dtype=jnp.int32) + i
        tv = jnp.full((L,), 0, dtype=jnp.int32) + t
        for l in range(CHUNK // L):
            brow = (t & (NIC - 1)) * CHUNK + l * L + lane
            c = plsc.load_gather(codes_v, [brow, iv])
            plsc.store_scatter(g_v, [tv, l * L + lane], c + iv * CBS)
        return carry

    lax.fori_loop(0, NGC, g_chunk, 0)

    def fire(t, carry):
        pltpu.async_copy(
            cb_hbm.at[g_v.at[t]],
            rows_v.at[t >> 2, pl.ds((t & (NIC - 1)) * CHUNK, CHUNK)], sem)
        return carry

    lax.fori_loop(0, NGC, fire, 0)

    def drain(t, carry):
        pltpu.make_async_copy(
            cb_hbm.at[g_v.at[t]],
            rows_v.at[t >> 2, pl.ds((t & (NIC - 1)) * CHUNK, CHUNK)],
            sem).wait()
        return carry

    lax.fori_loop(0, NGC, drain, 0)

    base = wid * BPW
    for i in range(NCB):
        pltpu.sync_copy(rows_v.at[i],
                        out_hbm.at[pl.ds(base, BPW), pl.ds(i * SUB, SUB)])


_pq_decode = functools.partial(
    pl.kernel,
    out_type=jax.ShapeDtypeStruct((BATCH, DIM), jnp.float32),
    mesh=plsc.VectorSubcoreMesh(core_axis_name="c", subcore_axis_name="s"),
    compiler_params=pltpu.CompilerParams(needs_layout_passes=False,
                                         use_tc_tiling_on_sc=False),
    scratch_types=[
        pltpu.VMEM((NIC, CHUNK), jnp.int32),
        pltpu.VMEM((BPW, NCB), jnp.int32),
        pltpu.VMEM((NGC, CHUNK), jnp.int32),
        pltpu.VMEM((NCB, BPW, SUB), jnp.float32),
        pltpu.SemaphoreType.DMA,
    ],
)(_pq_body)


def kernel(indices, codebooks, codes):
    idx2 = indices.astype(jnp.int32).reshape(BATCH // CHUNK, CHUNK)
    cb_flat = codebooks.reshape(NCB * CBS, SUB)
    return _pq_decode(idx2, cb_flat, codes)


# speedup vs baseline: 2.5462x; 2.5462x over previous
import functools

import jax
import jax.numpy as jnp
from jax import lax
from jax.experimental import pallas as pl
from jax.experimental.pallas import tpu as pltpu
from jax.experimental.pallas import tpu_sc as plsc

NUM_EMB = 1_000_000
DIM = 128
NCB = 8
CBS = 256
SUB = 16
BATCH = 16384

_INFO = plsc.get_sparse_core_info()
NC, NS, L = _INFO.num_cores, _INFO.num_subcores, _INFO.num_lanes
NW = NC * NS
BPW = BATCH // NW
CHUNK = 128
NIC = BPW // CHUNK
NGC = BPW * NCB // CHUNK


def _pq_body(idx_hbm, cb_hbm, *rest):
    (c0, c1, c2, c3, c4, c5, c6, c7, out_hbm,
     idx_v, ct_v, g_v, rows_v, sem) = rest
    planes = (c0, c1, c2, c3, c4, c5, c6, c7)
    wid = lax.axis_index("s") * NC + lax.axis_index("c")

    pltpu.sync_copy(idx_hbm.at[pl.ds(wid * NIC, NIC)], idx_v)

    handles = []
    for i in range(NCB):
        for c in range(NIC):
            handles.append(
                pltpu.async_copy(planes[i].at[idx_v.at[c]],
                                 ct_v.at[i * NIC + c], sem))
    for h in handles:
        h.wait()

    def g_chunk(t, carry):
        off = (t >> 2) * CBS
        for l in range(CHUNK // L):
            g_v[t, pl.ds(l * L, L)] = ct_v[t, pl.ds(l * L, L)] + off
        return carry

    lax.fori_loop(0, NGC, g_chunk, 0)

    def fire(t, carry):
        pltpu.async_copy(
            cb_hbm.at[g_v.at[t]],
            rows_v.at[t >> 2, pl.ds((t & (NIC - 1)) * CHUNK, CHUNK)], sem)
        return carry

    lax.fori_loop(0, NGC, fire, 0)

    def drain(t, carry):
        pltpu.make_async_copy(
            cb_hbm.at[g_v.at[t]],
            rows_v.at[t >> 2, pl.ds((t & (NIC - 1)) * CHUNK, CHUNK)],
            sem).wait()
        return carry

    lax.fori_loop(0, NGC, drain, 0)

    base = wid * BPW
    for i in range(NCB):
        pltpu.sync_copy(rows_v.at[i],
                        out_hbm.at[pl.ds(base, BPW), pl.ds(i * SUB, SUB)])


_pq_decode = functools.partial(
    pl.kernel,
    out_type=jax.ShapeDtypeStruct((BATCH, DIM), jnp.float32),
    mesh=plsc.VectorSubcoreMesh(core_axis_name="c", subcore_axis_name="s"),
    compiler_params=pltpu.CompilerParams(needs_layout_passes=False,
                                         use_tc_tiling_on_sc=False),
    scratch_types=[
        pltpu.VMEM((NIC, CHUNK), jnp.int32),
        pltpu.VMEM((NGC, CHUNK), jnp.int32),
        pltpu.VMEM((NGC, CHUNK), jnp.int32),
        pltpu.VMEM((NCB, BPW, SUB), jnp.float32),
        pltpu.SemaphoreType.DMA,
    ],
)(_pq_body)


def kernel(indices, codebooks, codes):
    idx2 = indices.astype(jnp.int32).reshape(BATCH // CHUNK, CHUNK)
    cb_flat = codebooks.reshape(NCB * CBS, SUB)
    planes = [codes[:, i] for i in range(NCB)]
    return _pq_decode(idx2, cb_flat, *planes)


# speedup vs baseline: 3.2360x; 1.2709x over previous
import functools

import jax
import jax.numpy as jnp
from jax import lax
from jax.experimental import pallas as pl
from jax.experimental.pallas import tpu as pltpu
from jax.experimental.pallas import tpu_sc as plsc

NUM_EMB = 1_000_000
DIM = 128
NCB = 8
CBS = 256
SUB = 16
BATCH = 16384

_INFO = plsc.get_sparse_core_info()
NC, NS, L = _INFO.num_cores, _INFO.num_subcores, _INFO.num_lanes
NW = NC * NS
BPW = BATCH // NW
CHUNK = 128
NIC = BPW // CHUNK
NGC = BPW * NCB // CHUNK


def _pq_body(idx_hbm, cb_hbm, lo_hbm, hi_hbm, out_hbm,
             idx_v, lo_v, hi_v, g_v, rows_v, sem):
    wid = lax.axis_index("s") * NC + lax.axis_index("c")

    pltpu.sync_copy(idx_hbm.at[pl.ds(wid * NIC, NIC)], idx_v)

    handles = []
    for c in range(NIC):
        handles.append(pltpu.async_copy(lo_hbm.at[idx_v.at[c]],
                                        lo_v.at[c], sem))
        handles.append(pltpu.async_copy(hi_hbm.at[idx_v.at[c]],
                                        hi_v.at[c], sem))
    for h in handles:
        h.wait()

    def g_chunk(c, carry):
        for i in range(NCB):
            src = lo_v if i < 4 else hi_v
            sh = (i & 3) * 8
            for l in range(CHUNK // L):
                w = src[c, pl.ds(l * L, L)]
                code = lax.shift_right_logical(w, sh) & 255
                g_v[i * NIC + c, pl.ds(l * L, L)] = code + i * CBS
        return carry

    lax.fori_loop(0, NIC, g_chunk, 0)

    def fire(t, carry):
        pltpu.async_copy(
            cb_hbm.at[g_v.at[t]],
            rows_v.at[t >> 2, pl.ds((t & (NIC - 1)) * CHUNK, CHUNK)], sem)
        return carry

    lax.fori_loop(0, NGC, fire, 0)

    def drain(t, carry):
        pltpu.make_async_copy(
            cb_hbm.at[g_v.at[t]],
            rows_v.at[t >> 2, pl.ds((t & (NIC - 1)) * CHUNK, CHUNK)],
            sem).wait()
        return carry

    lax.fori_loop(0, NGC, drain, 0)

    base = wid * BPW
    for i in range(NCB):
        pltpu.sync_copy(rows_v.at[i],
                        out_hbm.at[pl.ds(base, BPW), pl.ds(i * SUB, SUB)])


_pq_decode = functools.partial(
    pl.kernel,
    out_type=jax.ShapeDtypeStruct((BATCH, DIM), jnp.float32),
    mesh=plsc.VectorSubcoreMesh(core_axis_name="c", subcore_axis_name="s"),
    compiler_params=pltpu.CompilerParams(needs_layout_passes=False,
                                         use_tc_tiling_on_sc=False),
    scratch_types=[
        pltpu.VMEM((NIC, CHUNK), jnp.int32),
        pltpu.VMEM((NIC, CHUNK), jnp.int32),
        pltpu.VMEM((NIC, CHUNK), jnp.int32),
        pltpu.VMEM((NGC, CHUNK), jnp.int32),
        pltpu.VMEM((NCB, BPW, SUB), jnp.float32),
        pltpu.SemaphoreType.DMA,
    ],
)(_pq_body)


def kernel(indices, codebooks, codes):
    idx2 = indices.astype(jnp.int32).reshape(BATCH // CHUNK, CHUNK)
    cb_flat = codebooks.reshape(NCB * CBS, SUB)
    c = codes.astype(jnp.uint32)
    lo = (c[:, 0] | (c[:, 1] << 8) | (c[:, 2] << 16) | (c[:, 3] << 24))
    hi = (c[:, 4] | (c[:, 5] << 8) | (c[:, 6] << 16) | (c[:, 7] << 24))
    return _pq_decode(idx2, cb_flat, lo.astype(jnp.int32),
                      hi.astype(jnp.int32))


# speedup vs baseline: 3.8120x; 1.1780x over previous
import functools

import jax
import jax.numpy as jnp
from jax import lax
from jax.experimental import pallas as pl
from jax.experimental.pallas import tpu as pltpu
from jax.experimental.pallas import tpu_sc as plsc

NUM_EMB = 1_000_000
DIM = 128
NCB = 8
CBS = 256
SUB = 16
BATCH = 16384

_INFO = plsc.get_sparse_core_info()
NC, NS, L = _INFO.num_cores, _INFO.num_subcores, _INFO.num_lanes
NW = NC * NS
BPW = BATCH // NW
CHUNK = 128
NIC = BPW // CHUNK
NGC = BPW * NCB // CHUNK


def _pq_body(idx_hbm, cb_hbm, lo_hbm, hi_hbm, out_hbm,
             idx_v, lo_v, hi_v, g_v, rows_v, sem):
    wid = lax.axis_index("s") * NC + lax.axis_index("c")

    pltpu.sync_copy(idx_hbm.at[pl.ds(wid * NIC, NIC)], idx_v)

    handles = []
    for c in range(NIC):
        handles.append(pltpu.async_copy(lo_hbm.at[idx_v.at[c]],
                                        lo_v.at[c], sem))
        handles.append(pltpu.async_copy(hi_hbm.at[idx_v.at[c]],
                                        hi_v.at[c], sem))
    for h in handles:
        h.wait()

    def g_chunk(c, carry):
        for i in range(NCB):
            src = lo_v if i < 4 else hi_v
            sh = (i & 3) * 8
            for l in range(CHUNK // L):
                w = src[c, pl.ds(l * L, L)]
                code = lax.shift_right_logical(w, sh) & 255
                g_v[i * NIC + c, pl.ds(l * L, L)] = code + i * CBS
        return carry

    lax.fori_loop(0, NIC, g_chunk, 0)

    def fire(t, carry):
        pltpu.async_copy(
            cb_hbm.at[g_v.at[t]],
            rows_v.at[t >> 2, pl.ds((t & (NIC - 1)) * CHUNK, CHUNK)], sem)
        return carry

    lax.fori_loop(0, NGC, fire, 0)

    def drain(t, carry):
        pltpu.make_async_copy(
            cb_hbm.at[g_v.at[t]],
            rows_v.at[t >> 2, pl.ds((t & (NIC - 1)) * CHUNK, CHUNK)],
            sem).wait()
        return carry

    lax.fori_loop(0, NGC, drain, 0)

    base = wid * BPW
    for i in range(NCB):
        pltpu.sync_copy(rows_v.at[i],
                        out_hbm.at[pl.ds(base, BPW), pl.ds(i * SUB, SUB)])


_pq_decode = functools.partial(
    pl.kernel,
    out_type=jax.ShapeDtypeStruct((BATCH, DIM), jnp.float32),
    mesh=plsc.VectorSubcoreMesh(core_axis_name="c", subcore_axis_name="s"),
    compiler_params=pltpu.CompilerParams(needs_layout_passes=False,
                                         use_tc_tiling_on_sc=False),
    scratch_types=[
        pltpu.VMEM((NIC, CHUNK), jnp.int32),
        pltpu.VMEM((NIC, CHUNK), jnp.int32),
        pltpu.VMEM((NIC, CHUNK), jnp.int32),
        pltpu.VMEM((NGC, CHUNK), jnp.int32),
        pltpu.VMEM((NCB, BPW, SUB), jnp.float32),
        pltpu.SemaphoreType.DMA,
    ],
)(_pq_body)


def kernel(indices, codebooks, codes):
    idx2 = indices.astype(jnp.int32).reshape(BATCH // CHUNK, CHUNK)
    cb_flat = codebooks.reshape(NCB * CBS, SUB)
    lo = (codes[:, 0] | (codes[:, 1] << 8) | (codes[:, 2] << 16)
          | (codes[:, 3] << 24))
    hi = (codes[:, 4] | (codes[:, 5] << 8) | (codes[:, 6] << 16)
          | (codes[:, 7] << 24))
    return _pq_decode(idx2, cb_flat, lo, hi)


# speedup vs baseline: 4.9376x; 1.2953x over previous
import functools

import jax
import jax.numpy as jnp
from jax import lax
from jax.experimental import pallas as pl
from jax.experimental.pallas import tpu as pltpu
from jax.experimental.pallas import tpu_sc as plsc

NUM_EMB = 1_000_000
DIM = 128
NCB = 8
CBS = 256
SUB = 16
BATCH = 16384

_INFO = plsc.get_sparse_core_info()
NC, NS, L = _INFO.num_cores, _INFO.num_subcores, _INFO.num_lanes
NW = NC * NS
BPW = BATCH // NW
CHUNK = 128
NIC = BPW // CHUNK
NGC = BPW * NCB // CHUNK

PCG = 512
PSTEPS = 61
PPW = PCG * PSTEPS
PMAIN = PPW * NW
PTAIL = NUM_EMB - PMAIN


def _pack16(in_ref, c0, src, lo_ref, hi_ref, dst):
    w = [in_ref[c0, i, src] for i in range(NCB)]
    lo_ref[dst] = w[0] | (w[1] << 8) | (w[2] << 16) | (w[3] << 24)
    hi_ref[dst] = w[4] | (w[5] << 8) | (w[6] << 16) | (w[7] << 24)


def _pack_body(ct_hbm, tail_hbm, lo_hbm, hi_hbm, in_v, lo_v, hi_v,
               sem_in, sem_out):
    wid = lax.axis_index("s") * NC + lax.axis_index("c")
    w0 = wid * PPW

    pltpu.async_copy(ct_hbm.at[:, pl.ds(w0, PCG)], in_v.at[0], sem_in)

    def step(c, carry):
        buf = c & 1
        e0 = w0 + c * PCG
        pltpu.make_async_copy(ct_hbm.at[:, pl.ds(e0, PCG)],
                              in_v.at[buf], sem_in).wait()

        @pl.when(c + 1 < PSTEPS)
        def _():
            pltpu.async_copy(ct_hbm.at[:, pl.ds(e0 + PCG, PCG)],
                             in_v.at[1 - buf], sem_in)

        @pl.when(c >= 2)
        def _():
            o0 = w0 + (c - 2) * PCG
            pltpu.make_async_copy(lo_v.at[buf],
                                  lo_hbm.at[pl.ds(o0, PCG)], sem_out).wait()
            pltpu.make_async_copy(hi_v.at[buf],
                                  hi_hbm.at[pl.ds(o0, PCG)], sem_out).wait()

        for k in range(PCG // L):
            _pack16(in_v, buf, pl.ds(k * L, L), lo_v, hi_v,
                    (buf, pl.ds(k * L, L)))
        pltpu.async_copy(lo_v.at[buf], lo_hbm.at[pl.ds(e0, PCG)], sem_out)
        pltpu.async_copy(hi_v.at[buf], hi_hbm.at[pl.ds(e0, PCG)], sem_out)
        return carry

    lax.fori_loop(0, PSTEPS, step, 0)
    for c in (PSTEPS - 2, PSTEPS - 1):
        o0 = w0 + c * PCG
        pltpu.make_async_copy(lo_v.at[c & 1],
                              lo_hbm.at[pl.ds(o0, PCG)], sem_out).wait()
        pltpu.make_async_copy(hi_v.at[c & 1],
                              hi_hbm.at[pl.ds(o0, PCG)], sem_out).wait()

    @pl.when(wid == NW - 1)
    def _():
        for off, sz, out_sz in ((0, PCG, PCG), (PCG, 128, PTAIL - PCG)):
            pltpu.sync_copy(tail_hbm.at[:, pl.ds(off, sz)],
                            in_v.at[0, :, pl.ds(0, sz)])
            for k in range(sz // L):
                _pack16(in_v, 0, pl.ds(k * L, L), lo_v, hi_v,
                        (0, pl.ds(k * L, L)))
            pltpu.sync_copy(lo_v.at[0, pl.ds(0, out_sz)],
                            lo_hbm.at[pl.ds(PMAIN + off, out_sz)])
            pltpu.sync_copy(hi_v.at[0, pl.ds(0, out_sz)],
                            hi_hbm.at[pl.ds(PMAIN + off, out_sz)])


_pack = functools.partial(
    pl.kernel,
    out_type=(jax.ShapeDtypeStruct((NUM_EMB,), jnp.int32),
              jax.ShapeDtypeStruct((NUM_EMB,), jnp.int32)),
    mesh=plsc.VectorSubcoreMesh(core_axis_name="c", subcore_axis_name="s"),
    compiler_params=pltpu.CompilerParams(use_tc_tiling_on_sc=True),
    scratch_types=[
        pltpu.VMEM((2, NCB, PCG), jnp.int32),
        pltpu.VMEM((2, PCG), jnp.int32),
        pltpu.VMEM((2, PCG), jnp.int32),
        pltpu.SemaphoreType.DMA,
        pltpu.SemaphoreType.DMA,
    ],
)(_pack_body)


def _pq_body(idx_hbm, cb_hbm, lo_hbm, hi_hbm, out_hbm,
             idx_v, lo_v, hi_v, g_v, rows_v, sem):
    wid = lax.axis_index("s") * NC + lax.axis_index("c")

    pltpu.sync_copy(idx_hbm.at[pl.ds(wid * NIC, NIC)], idx_v)

    handles = []
    for c in range(NIC):
        handles.append(pltpu.async_copy(lo_hbm.at[idx_v.at[c]],
                                        lo_v.at[c], sem))
        handles.append(pltpu.async_copy(hi_hbm.at[idx_v.at[c]],
                                        hi_v.at[c], sem))
    for h in handles:
        h.wait()

    def g_chunk(c, carry):
        for i in range(NCB):
            src = lo_v if i < 4 else hi_v
            sh = (i & 3) * 8
            for l in range(CHUNK // L):
                w = src[c, pl.ds(l * L, L)]
                code = lax.shift_right_logical(w, sh) & 255
                g_v[i * NIC + c, pl.ds(l * L, L)] = code + i * CBS
        return carry

    lax.fori_loop(0, NIC, g_chunk, 0)

    def fire(t, carry):
        pltpu.async_copy(
            cb_hbm.at[g_v.at[t]],
            rows_v.at[t >> 2, pl.ds((t & (NIC - 1)) * CHUNK, CHUNK)], sem)
        return carry

    lax.fori_loop(0, NGC, fire, 0)

    def drain(t, carry):
        pltpu.make_async_copy(
            cb_hbm.at[g_v.at[t]],
            rows_v.at[t >> 2, pl.ds((t & (NIC - 1)) * CHUNK, CHUNK)],
            sem).wait()
        return carry

    lax.fori_loop(0, NGC, drain, 0)

    base = wid * BPW
    for i in range(NCB):
        pltpu.sync_copy(rows_v.at[i],
                        out_hbm.at[pl.ds(base, BPW), pl.ds(i * SUB, SUB)])


_pq_decode = functools.partial(
    pl.kernel,
    out_type=jax.ShapeDtypeStruct((BATCH, DIM), jnp.float32),
    mesh=plsc.VectorSubcoreMesh(core_axis_name="c", subcore_axis_name="s"),
    compiler_params=pltpu.CompilerParams(needs_layout_passes=False,
                                         use_tc_tiling_on_sc=False),
    scratch_types=[
        pltpu.VMEM((NIC, CHUNK), jnp.int32),
        pltpu.VMEM((NIC, CHUNK), jnp.int32),
        pltpu.VMEM((NIC, CHUNK), jnp.int32),
        pltpu.VMEM((NGC, CHUNK), jnp.int32),
        pltpu.VMEM((NCB, BPW, SUB), jnp.float32),
        pltpu.SemaphoreType.DMA,
    ],
)(_pq_body)


def kernel(indices, codebooks, codes):
    idx2 = indices.astype(jnp.int32).reshape(BATCH // CHUNK, CHUNK)
    cb_flat = codebooks.reshape(NCB * CBS, SUB)
    tail = jnp.pad(codes[PMAIN:, :].T, ((0, 0), (0, PCG + 128 - PTAIL)))
    lo, hi = _pack(codes.T, tail)
    return _pq_decode(idx2, cb_flat, lo, hi)


# speedup vs baseline: 5.5365x; 1.1213x over previous
import functools

import jax
import jax.numpy as jnp
from jax import lax
from jax.experimental import pallas as pl
from jax.experimental.pallas import tpu as pltpu
from jax.experimental.pallas import tpu_sc as plsc

NUM_EMB = 1_000_000
DIM = 128
NCB = 8
CBS = 256
SUB = 16
BATCH = 16384

_INFO = plsc.get_sparse_core_info()
NC, NS, L = _INFO.num_cores, _INFO.num_subcores, _INFO.num_lanes
NW = NC * NS
BPW = BATCH // NW
CHUNK = 128
NIC = BPW // CHUNK
NGC = BPW * NCB // CHUNK

PCG = 512
PSTEPS = 61
PPW = PCG * PSTEPS
PMAIN = PPW * NW
PTAIL = NUM_EMB - PMAIN


def _pack16(in_ref, src, lo_ref, hi_ref, dst):
    w = [in_ref[i, src] for i in range(NCB)]
    lo_ref[dst] = w[0] | (w[1] << 8) | (w[2] << 16) | (w[3] << 24)
    hi_ref[dst] = w[4] | (w[5] << 8) | (w[6] << 16) | (w[7] << 24)


def _pack_body(ct_hbm, tail_hbm, lo_hbm, hi_hbm, in0_v, in1_v, lo0_v, lo1_v,
               hi0_v, hi1_v, sem_in, sem_out):
    wid = lax.axis_index("s") * NC + lax.axis_index("c")
    w0 = wid * PPW
    bufs = ((in0_v, lo0_v, hi0_v), (in1_v, lo1_v, hi1_v))

    pltpu.async_copy(ct_hbm.at[:, pl.ds(w0, PCG)], in0_v, sem_in)

    def emit_chunk(c, buf):
        in_v, lo_v, hi_v = bufs[buf]
        e0 = w0 + c * PCG
        pltpu.make_async_copy(ct_hbm.at[:, pl.ds(e0, PCG)],
                              in_v, sem_in).wait()

        @pl.when(c >= 2)
        def _():
            o0 = w0 + (c - 2) * PCG
            pltpu.make_async_copy(lo_v, lo_hbm.at[pl.ds(o0, PCG)],
                                  sem_out).wait()
            pltpu.make_async_copy(hi_v, hi_hbm.at[pl.ds(o0, PCG)],
                                  sem_out).wait()

        for k in range(PCG // L):
            _pack16(in_v, pl.ds(k * L, L), lo_v, hi_v, pl.ds(k * L, L))

        @pl.when(c + 2 < PSTEPS)
        def _():
            pltpu.async_copy(ct_hbm.at[:, pl.ds(e0 + 2 * PCG, PCG)],
                             in_v, sem_in)

        pltpu.async_copy(lo_v, lo_hbm.at[pl.ds(e0, PCG)], sem_out)
        pltpu.async_copy(hi_v, hi_hbm.at[pl.ds(e0, PCG)], sem_out)

    pltpu.async_copy(ct_hbm.at[:, pl.ds(w0 + PCG, PCG)], in1_v, sem_in)

    def step(d, carry):
        emit_chunk(2 * d, 0)
        emit_chunk(2 * d + 1, 1)
        return carry

    lax.fori_loop(0, PSTEPS // 2, step, 0)
    emit_chunk(PSTEPS - 1, 0)

    for c, (_, lo_v, hi_v) in ((PSTEPS - 2, bufs[1]), (PSTEPS - 1, bufs[0])):
        o0 = w0 + c * PCG
        pltpu.make_async_copy(lo_v, lo_hbm.at[pl.ds(o0, PCG)],
                              sem_out).wait()
        pltpu.make_async_copy(hi_v, hi_hbm.at[pl.ds(o0, PCG)],
                              sem_out).wait()

    @pl.when(wid == NW - 1)
    def _():
        for off, sz, out_sz in ((0, PCG, PCG), (PCG, 128, PTAIL - PCG)):
            pltpu.sync_copy(tail_hbm.at[:, pl.ds(off, sz)],
                            in0_v.at[:, pl.ds(0, sz)])
            for k in range(sz // L):
                _pack16(in0_v, pl.ds(k * L, L), lo0_v, hi0_v,
                        pl.ds(k * L, L))
            pltpu.sync_copy(lo0_v.at[pl.ds(0, out_sz)],
                            lo_hbm.at[pl.ds(PMAIN + off, out_sz)])
            pltpu.sync_copy(hi0_v.at[pl.ds(0, out_sz)],
                            hi_hbm.at[pl.ds(PMAIN + off, out_sz)])


_pack = functools.partial(
    pl.kernel,
    out_type=(jax.ShapeDtypeStruct((NUM_EMB,), jnp.int32),
              jax.ShapeDtypeStruct((NUM_EMB,), jnp.int32)),
    mesh=plsc.VectorSubcoreMesh(core_axis_name="c", subcore_axis_name="s"),
    compiler_params=pltpu.CompilerParams(use_tc_tiling_on_sc=True),
    scratch_types=[
        pltpu.VMEM((NCB, PCG), jnp.int32),
        pltpu.VMEM((NCB, PCG), jnp.int32),
        pltpu.VMEM((PCG,), jnp.int32),
        pltpu.VMEM((PCG,), jnp.int32),
        pltpu.VMEM((PCG,), jnp.int32),
        pltpu.VMEM((PCG,), jnp.int32),
        pltpu.SemaphoreType.DMA,
        pltpu.SemaphoreType.DMA,
    ],
)(_pack_body)


def _pq_body(idx_hbm, cb_hbm, lo_hbm, hi_hbm, out_hbm,
             idx_v, lo_v, hi_v, g_v, rows_v, sem):
    wid = lax.axis_index("s") * NC + lax.axis_index("c")

    pltpu.sync_copy(idx_hbm.at[pl.ds(wid * NIC, NIC)], idx_v)

    handles = []
    for c in range(NIC):
        handles.append(pltpu.async_copy(lo_hbm.at[idx_v.at[c]],
                                        lo_v.at[c], sem))
        handles.append(pltpu.async_copy(hi_hbm.at[idx_v.at[c]],
                                        hi_v.at[c], sem))
    for h in handles:
        h.wait()

    def g_chunk(c, carry):
        for i in range(NCB):
            src = lo_v if i < 4 else hi_v
            sh = (i & 3) * 8
            for l in range(CHUNK // L):
                w = src[c, pl.ds(l * L, L)]
                code = lax.shift_right_logical(w, sh) & 255
                g_v[i * NIC + c, pl.ds(l * L, L)] = code + i * CBS
        return carry

    lax.fori_loop(0, NIC, g_chunk, 0)

    def fire(t, carry):
        pltpu.async_copy(
            cb_hbm.at[g_v.at[t]],
            rows_v.at[t >> 2, pl.ds((t & (NIC - 1)) * CHUNK, CHUNK)], sem)
        return carry

    lax.fori_loop(0, NGC, fire, 0)

    def drain(t, carry):
        pltpu.make_async_copy(
            cb_hbm.at[g_v.at[t]],
            rows_v.at[t >> 2, pl.ds((t & (NIC - 1)) * CHUNK, CHUNK)],
            sem).wait()
        return carry

    lax.fori_loop(0, NGC, drain, 0)

    base = wid * BPW
    for i in range(NCB):
        pltpu.sync_copy(rows_v.at[i],
                        out_hbm.at[pl.ds(base, BPW), pl.ds(i * SUB, SUB)])


_pq_decode = functools.partial(
    pl.kernel,
    out_type=jax.ShapeDtypeStruct((BATCH, DIM), jnp.float32),
    mesh=plsc.VectorSubcoreMesh(core_axis_name="c", subcore_axis_name="s"),
    compiler_params=pltpu.CompilerParams(needs_layout_passes=False,
                                         use_tc_tiling_on_sc=False),
    scratch_types=[
        pltpu.VMEM((NIC, CHUNK), jnp.int32),
        pltpu.VMEM((NIC, CHUNK), jnp.int32),
        pltpu.VMEM((NIC, CHUNK), jnp.int32),
        pltpu.VMEM((NGC, CHUNK), jnp.int32),
        pltpu.VMEM((NCB, BPW, SUB), jnp.float32),
        pltpu.SemaphoreType.DMA,
    ],
)(_pq_body)


def kernel(indices, codebooks, codes):
    idx2 = indices.astype(jnp.int32).reshape(BATCH // CHUNK, CHUNK)
    cb_flat = codebooks.reshape(NCB * CBS, SUB)
    tail = jnp.pad(codes[PMAIN:, :].T, ((0, 0), (0, PCG + 128 - PTAIL)))
    lo, hi = _pack(codes.T, tail)
    return _pq_decode(idx2, cb_flat, lo, hi)


# speedup vs baseline: 5.5616x; 1.0045x over previous
import functools

import jax
import jax.numpy as jnp
from jax import lax
from jax.experimental import pallas as pl
from jax.experimental.pallas import tpu as pltpu
from jax.experimental.pallas import tpu_sc as plsc

NUM_EMB = 1_000_000
DIM = 128
NCB = 8
CBS = 256
SUB = 16
BATCH = 16384

_INFO = plsc.get_sparse_core_info()
NC, NS, L = _INFO.num_cores, _INFO.num_subcores, _INFO.num_lanes
NW = NC * NS
BPW = BATCH // NW
CHUNK = 128
NIC = BPW // CHUNK
NGC = BPW * NCB // CHUNK

PCG = 512
PSTEPS = 61
PPW = PCG * PSTEPS
PMAIN = PPW * NW
PTAIL = NUM_EMB - PMAIN


def _pack16(in_ref, src, lo_ref, hi_ref, dst):
    w = [in_ref[i, src] for i in range(NCB)]
    lo_ref[dst] = w[0] | (w[1] << 8) | (w[2] << 16) | (w[3] << 24)
    hi_ref[dst] = w[4] | (w[5] << 8) | (w[6] << 16) | (w[7] << 24)


def _pack_body(ct_hbm, tail_hbm, lo_hbm, hi_hbm, in0_v, in1_v, lo0_v, lo1_v,
               hi0_v, hi1_v, sem_in, sem_out):
    wid = lax.axis_index("s") * NC + lax.axis_index("c")
    w0 = wid * PPW
    bufs = ((in0_v, lo0_v, hi0_v), (in1_v, lo1_v, hi1_v))

    pltpu.async_copy(ct_hbm.at[:, pl.ds(w0, PCG)], in0_v, sem_in)

    def emit_chunk(c, buf):
        in_v, lo_v, hi_v = bufs[buf]
        e0 = w0 + c * PCG
        pltpu.make_async_copy(ct_hbm.at[:, pl.ds(e0, PCG)],
                              in_v, sem_in).wait()

        @pl.when(c >= 2)
        def _():
            o0 = w0 + (c - 2) * PCG
            pltpu.make_async_copy(lo_v, lo_hbm.at[pl.ds(o0, PCG)],
                                  sem_out).wait()
            pltpu.make_async_copy(hi_v, hi_hbm.at[pl.ds(o0, PCG)],
                                  sem_out).wait()

        for k in range(PCG // L):
            _pack16(in_v, pl.ds(k * L, L), lo_v, hi_v, pl.ds(k * L, L))

        @pl.when(c + 2 < PSTEPS)
        def _():
            pltpu.async_copy(ct_hbm.at[:, pl.ds(e0 + 2 * PCG, PCG)],
                             in_v, sem_in)

        pltpu.async_copy(lo_v, lo_hbm.at[pl.ds(e0, PCG)], sem_out)
        pltpu.async_copy(hi_v, hi_hbm.at[pl.ds(e0, PCG)], sem_out)

    pltpu.async_copy(ct_hbm.at[:, pl.ds(w0 + PCG, PCG)], in1_v, sem_in)

    def step(d, carry):
        emit_chunk(2 * d, 0)
        emit_chunk(2 * d + 1, 1)
        return carry

    lax.fori_loop(0, PSTEPS // 2, step, 0)
    emit_chunk(PSTEPS - 1, 0)

    for c, (_, lo_v, hi_v) in ((PSTEPS - 2, bufs[1]), (PSTEPS - 1, bufs[0])):
        o0 = w0 + c * PCG
        pltpu.make_async_copy(lo_v, lo_hbm.at[pl.ds(o0, PCG)],
                              sem_out).wait()
        pltpu.make_async_copy(hi_v, hi_hbm.at[pl.ds(o0, PCG)],
                              sem_out).wait()

    @pl.when(wid == NW - 1)
    def _():
        for off, sz, out_sz in ((0, PCG, PCG), (PCG, 128, PTAIL - PCG)):
            pltpu.sync_copy(tail_hbm.at[:, pl.ds(off, sz)],
                            in0_v.at[:, pl.ds(0, sz)])
            for k in range(sz // L):
                _pack16(in0_v, pl.ds(k * L, L), lo0_v, hi0_v,
                        pl.ds(k * L, L))
            pltpu.sync_copy(lo0_v.at[pl.ds(0, out_sz)],
                            lo_hbm.at[pl.ds(PMAIN + off, out_sz)])
            pltpu.sync_copy(hi0_v.at[pl.ds(0, out_sz)],
                            hi_hbm.at[pl.ds(PMAIN + off, out_sz)])


_pack = functools.partial(
    pl.kernel,
    out_type=(jax.ShapeDtypeStruct((NUM_EMB,), jnp.int32),
              jax.ShapeDtypeStruct((NUM_EMB,), jnp.int32)),
    mesh=plsc.VectorSubcoreMesh(core_axis_name="c", subcore_axis_name="s"),
    compiler_params=pltpu.CompilerParams(use_tc_tiling_on_sc=True),
    scratch_types=[
        pltpu.VMEM((NCB, PCG), jnp.int32),
        pltpu.VMEM((NCB, PCG), jnp.int32),
        pltpu.VMEM((PCG,), jnp.int32),
        pltpu.VMEM((PCG,), jnp.int32),
        pltpu.VMEM((PCG,), jnp.int32),
        pltpu.VMEM((PCG,), jnp.int32),
        pltpu.SemaphoreType.DMA,
        pltpu.SemaphoreType.DMA,
    ],
)(_pack_body)


def _pq_body(idx_hbm, cb_hbm, lo_hbm, hi_hbm, out_hbm,
             idx_v, lo_v, hi_v, g_v, rows_v, sem, out_sem):
    wid = lax.axis_index("s") * NC + lax.axis_index("c")

    pltpu.sync_copy(idx_hbm.at[pl.ds(wid * NIC, NIC)], idx_v)

    handles = []
    for c in range(NIC):
        handles.append(pltpu.async_copy(lo_hbm.at[idx_v.at[c]],
                                        lo_v.at[c], sem))
        handles.append(pltpu.async_copy(hi_hbm.at[idx_v.at[c]],
                                        hi_v.at[c], sem))
    for h in handles:
        h.wait()

    def g_chunk(c, carry):
        for i in range(NCB):
            src = lo_v if i < 4 else hi_v
            sh = (i & 3) * 8
            for l in range(CHUNK // L):
                w = src[c, pl.ds(l * L, L)]
                code = lax.shift_right_logical(w, sh) & 255
                g_v[i * NIC + c, pl.ds(l * L, L)] = code + i * CBS
        return carry

    lax.fori_loop(0, NIC, g_chunk, 0)

    def fire(t, carry):
        pltpu.async_copy(
            cb_hbm.at[g_v.at[t]],
            rows_v.at[t >> 2, pl.ds((t & (NIC - 1)) * CHUNK, CHUNK)], sem)
        return carry

    lax.fori_loop(0, NGC, fire, 0)

    base = wid * BPW
    out_handles = []
    for i in range(NCB):
        for c in range(NIC):
            pltpu.make_async_copy(
                cb_hbm.at[g_v.at[i * NIC + c]],
                rows_v.at[i, pl.ds(c * CHUNK, CHUNK)], sem).wait()
        out_handles.append(pltpu.async_copy(
            rows_v.at[i],
            out_hbm.at[pl.ds(base, BPW), pl.ds(i * SUB, SUB)], out_sem))
    for h in out_handles:
        h.wait()


_pq_decode = functools.partial(
    pl.kernel,
    out_type=jax.ShapeDtypeStruct((BATCH, DIM), jnp.float32),
    mesh=plsc.VectorSubcoreMesh(core_axis_name="c", subcore_axis_name="s"),
    compiler_params=pltpu.CompilerParams(needs_layout_passes=False,
                                         use_tc_tiling_on_sc=False),
    scratch_types=[
        pltpu.VMEM((NIC, CHUNK), jnp.int32),
        pltpu.VMEM((NIC, CHUNK), jnp.int32),
        pltpu.VMEM((NIC, CHUNK), jnp.int32),
        pltpu.VMEM((NGC, CHUNK), jnp.int32),
        pltpu.VMEM((NCB, BPW, SUB), jnp.float32),
        pltpu.SemaphoreType.DMA,
        pltpu.SemaphoreType.DMA,
    ],
)(_pq_body)


def kernel(indices, codebooks, codes):
    idx2 = indices.astype(jnp.int32).reshape(BATCH // CHUNK, CHUNK)
    cb_flat = codebooks.reshape(NCB * CBS, SUB)
    tail = jnp.pad(codes[PMAIN:, :].T, ((0, 0), (0, PCG + 128 - PTAIL)))
    lo, hi = _pack(codes.T, tail)
    return _pq_decode(idx2, cb_flat, lo, hi)


# speedup vs baseline: 6.4141x; 1.1533x over previous
import functools

import jax
import jax.numpy as jnp
from jax import lax
from jax.experimental import pallas as pl
from jax.experimental.pallas import tpu as pltpu
from jax.experimental.pallas import tpu_sc as plsc

NUM_EMB = 1_000_000
DIM = 128
NCB = 8
CBS = 256
SUB = 16
BATCH = 16384

_INFO = plsc.get_sparse_core_info()
NC, NS, L = _INFO.num_cores, _INFO.num_subcores, _INFO.num_lanes
NW = NC * NS
BPW = BATCH // NW
CHUNK = 128
NIC = BPW // CHUNK
NGC = BPW * NCB // CHUNK

PCG = 512
PSTEPS = 61
PPW = PCG * PSTEPS
PMAIN = PPW * NW
PTAIL = NUM_EMB - PMAIN


def _pack16(in_ref, src, lo_ref, hi_ref, dst):
    w = [in_ref[i, src] for i in range(NCB)]
    lo_ref[dst] = w[0] | (w[1] << 8) | (w[2] << 16) | (w[3] << 24)
    hi_ref[dst] = w[4] | (w[5] << 8) | (w[6] << 16) | (w[7] << 24)


NBUF = 4


def _pack_body(ct_hbm, tail_hbm, lo_hbm, hi_hbm, *bufrefs):
    (in0_v, in1_v, in2_v, in3_v, lo0_v, lo1_v, lo2_v, lo3_v,
     hi0_v, hi1_v, hi2_v, hi3_v, sem_in, sem_out) = bufrefs
    wid = lax.axis_index("s") * NC + lax.axis_index("c")
    w0 = wid * PPW
    bufs = ((in0_v, lo0_v, hi0_v), (in1_v, lo1_v, hi1_v),
            (in2_v, lo2_v, hi2_v), (in3_v, lo3_v, hi3_v))

    for b in range(NBUF):
        pltpu.async_copy(ct_hbm.at[:, pl.ds(w0 + b * PCG, PCG)],
                         bufs[b][0], sem_in)

    def emit_chunk(c, buf):
        in_v, lo_v, hi_v = bufs[buf]
        e0 = w0 + c * PCG
        pltpu.make_async_copy(ct_hbm.at[:, pl.ds(e0, PCG)],
                              in_v, sem_in).wait()

        @pl.when(c >= NBUF)
        def _():
            o0 = w0 + (c - NBUF) * PCG
            pltpu.make_async_copy(lo_v, lo_hbm.at[pl.ds(o0, PCG)],
                                  sem_out).wait()
            pltpu.make_async_copy(hi_v, hi_hbm.at[pl.ds(o0, PCG)],
                                  sem_out).wait()

        for k in range(PCG // L):
            _pack16(in_v, pl.ds(k * L, L), lo_v, hi_v, pl.ds(k * L, L))

        @pl.when(c + NBUF < PSTEPS)
        def _():
            pltpu.async_copy(ct_hbm.at[:, pl.ds(e0 + NBUF * PCG, PCG)],
                             in_v, sem_in)

        pltpu.async_copy(lo_v, lo_hbm.at[pl.ds(e0, PCG)], sem_out)
        pltpu.async_copy(hi_v, hi_hbm.at[pl.ds(e0, PCG)], sem_out)

    def step(d, carry):
        for b in range(NBUF):
            emit_chunk(NBUF * d + b, b)
        return carry

    lax.fori_loop(0, PSTEPS // NBUF, step, 0)
    emit_chunk(PSTEPS - 1, (PSTEPS - 1) % NBUF)

    for c in range(PSTEPS - NBUF, PSTEPS):
        _, lo_v, hi_v = bufs[c % NBUF]
        o0 = w0 + c * PCG
        pltpu.make_async_copy(lo_v, lo_hbm.at[pl.ds(o0, PCG)],
                              sem_out).wait()
        pltpu.make_async_copy(hi_v, hi_hbm.at[pl.ds(o0, PCG)],
                              sem_out).wait()

    @pl.when(wid == NW - 1)
    def _():
        for off, sz, out_sz in ((0, PCG, PCG), (PCG, 128, PTAIL - PCG)):
            pltpu.sync_copy(tail_hbm.at[:, pl.ds(off, sz)],
                            in0_v.at[:, pl.ds(0, sz)])
            for k in range(sz // L):
                _pack16(in0_v, pl.ds(k * L, L), lo0_v, hi0_v,
                        pl.ds(k * L, L))
            pltpu.sync_copy(lo0_v.at[pl.ds(0, out_sz)],
                            lo_hbm.at[pl.ds(PMAIN + off, out_sz)])
            pltpu.sync_copy(hi0_v.at[pl.ds(0, out_sz)],
                            hi_hbm.at[pl.ds(PMAIN + off, out_sz)])


_pack = functools.partial(
    pl.kernel,
    out_type=(jax.ShapeDtypeStruct((NUM_EMB,), jnp.int32),
              jax.ShapeDtypeStruct((NUM_EMB,), jnp.int32)),
    mesh=plsc.VectorSubcoreMesh(core_axis_name="c", subcore_axis_name="s"),
    compiler_params=pltpu.CompilerParams(use_tc_tiling_on_sc=True),
    scratch_types=(
        [pltpu.VMEM((NCB, PCG), jnp.int32)] * NBUF
        + [pltpu.VMEM((PCG,), jnp.int32)] * (2 * NBUF)
        + [pltpu.SemaphoreType.DMA, pltpu.SemaphoreType.DMA]
    ),
)(_pack_body)


def _pq_body(idx_hbm, cb_hbm, lo_hbm, hi_hbm, out_hbm,
             idx_v, lo_v, hi_v, g_v, rows_v, sem, out_sem):
    wid = lax.axis_index("s") * NC + lax.axis_index("c")

    pltpu.sync_copy(idx_hbm.at[pl.ds(wid * NIC, NIC)], idx_v)

    handles = []
    for c in range(NIC):
        handles.append(pltpu.async_copy(lo_hbm.at[idx_v.at[c]],
                                        lo_v.at[c], sem))
        handles.append(pltpu.async_copy(hi_hbm.at[idx_v.at[c]],
                                        hi_v.at[c], sem))
    for h in handles:
        h.wait()

    def g_chunk(c, carry):
        for i in range(NCB):
            src = lo_v if i < 4 else hi_v
            sh = (i & 3) * 8
            for l in range(CHUNK // L):
                w = src[c, pl.ds(l * L, L)]
                code = lax.shift_right_logical(w, sh) & 255
                g_v[i * NIC + c, pl.ds(l * L, L)] = code + i * CBS
        return carry

    lax.fori_loop(0, NIC, g_chunk, 0)

    def fire(t, carry):
        pltpu.async_copy(
            cb_hbm.at[g_v.at[t]],
            rows_v.at[t >> 2, pl.ds((t & (NIC - 1)) * CHUNK, CHUNK)], sem)
        return carry

    lax.fori_loop(0, NGC, fire, 0)

    base = wid * BPW
    out_handles = []
    for i in range(NCB):
        for c in range(NIC):
            pltpu.make_async_copy(
                cb_hbm.at[g_v.at[i * NIC + c]],
                rows_v.at[i, pl.ds(c * CHUNK, CHUNK)], sem).wait()
        out_handles.append(pltpu.async_copy(
            rows_v.at[i],
            out_hbm.at[pl.ds(base, BPW), pl.ds(i * SUB, SUB)], out_sem))
    for h in out_handles:
        h.wait()


_pq_decode = functools.partial(
    pl.kernel,
    out_type=jax.ShapeDtypeStruct((BATCH, DIM), jnp.float32),
    mesh=plsc.VectorSubcoreMesh(core_axis_name="c", subcore_axis_name="s"),
    compiler_params=pltpu.CompilerParams(needs_layout_passes=False,
                                         use_tc_tiling_on_sc=False),
    scratch_types=[
        pltpu.VMEM((NIC, CHUNK), jnp.int32),
        pltpu.VMEM((NIC, CHUNK), jnp.int32),
        pltpu.VMEM((NIC, CHUNK), jnp.int32),
        pltpu.VMEM((NGC, CHUNK), jnp.int32),
        pltpu.VMEM((NCB, BPW, SUB), jnp.float32),
        pltpu.SemaphoreType.DMA,
        pltpu.SemaphoreType.DMA,
    ],
)(_pq_body)


def kernel(indices, codebooks, codes):
    idx2 = indices.astype(jnp.int32).reshape(BATCH // CHUNK, CHUNK)
    cb_flat = codebooks.reshape(NCB * CBS, SUB)
    tail = jnp.pad(codes[PMAIN:, :].T, ((0, 0), (0, PCG + 128 - PTAIL)))
    lo, hi = _pack(codes.T, tail)
    return _pq_decode(idx2, cb_flat, lo, hi)
